# baseline (speedup 1.0000x reference)
import functools

import jax
import jax.numpy as jnp
from jax import lax
from jax.experimental import pallas as pl
from jax.experimental.pallas import tpu as pltpu
from jax.experimental.pallas import tpu_sc as plsc

_NC = 2
_NS = 16
_NW = _NC * _NS
_L = 16
_BB = 128
_EMB = 64


def _sc_embedding_lookup(ids4, table, b, s):
    nbt = b // _BB
    nst = s // 8
    assert nbt == _NW and nst * 8 == s and s % 2 == 0

    mesh = plsc.VectorSubcoreMesh(core_axis_name="c", subcore_axis_name="s")

    @functools.partial(
        pl.kernel,
        mesh=mesh,
        out_type=jax.ShapeDtypeStruct((s, _EMB // 8, nbt, 8 * _BB), jnp.float32),
        scratch_types=[
            pltpu.VMEM((nst, 8, _BB), jnp.int32),
            pltpu.VMEM((2, _BB, _EMB), jnp.float32),
            pltpu.VMEM((2, _BB * _EMB), jnp.float32),
            pltpu.SemaphoreType.DMA,
            pltpu.SemaphoreType.DMA,
            pltpu.SemaphoreType.DMA,
            pltpu.SemaphoreType.DMA,
        ],
        compiler_params=pltpu.CompilerParams(
            use_tc_tiling_on_sc=False, needs_layout_passes=False
        ),
    )
    def body(ids_hbm, table_hbm, y_hbm, idx_v, rows_v, yblk_v, g0, g1, s0, s1):
        wid = lax.axis_index("s") * _NC + lax.axis_index("c")
        gsems = (g0, g1)
        ssems = (s0, s1)

        def start_gather(k, buf):
            st = k // 8
            sr = k % 8
            pltpu.async_copy(
                table_hbm.at[idx_v.at[st, sr]], rows_v.at[buf], gsems[buf]
            )

        def wait_gather(buf):
            pltpu.make_async_copy(
                table_hbm.at[pl.ds(0, _BB)], rows_v.at[buf], gsems[buf]
            ).wait()

        def start_store(k, buf):
            for tc in range(_EMB // 8):
                pltpu.async_copy(
                    yblk_v.at[buf, pl.ds(tc * 8 * _BB, 8 * _BB)],
                    y_hbm.at[k, tc, wid],
                    ssems[buf],
                )

        def wait_store(buf):
            for tc in range(_EMB // 8):
                pltpu.make_async_copy(
                    yblk_v.at[buf, pl.ds(tc * 8 * _BB, 8 * _BB)],
                    y_hbm.at[0, tc, 0],
                    ssems[buf],
                ).wait()

        def transpose(buf):
            iota = lax.iota(jnp.int32, _L)

            def tbody(c, carry):
                cvec = jnp.full((_L,), 0, jnp.int32) + c
                base = c * _BB
                for j in range(_BB // _L):
                    brv = iota + (j * _L)
                    vals = plsc.load_gather(rows_v.at[buf], [brv, cvec])
                    yblk_v[buf, pl.ds(base + j * _L, _L)] = vals
                return carry

            lax.fori_loop(0, _EMB, tbody, 0)

        for st in range(nst):
            pltpu.sync_copy(ids_hbm.at[st, wid], idx_v.at[st])

        nblk = s
        start_gather(0, 0)

        wait_gather(0)
        start_gather(1, 1)
        transpose(0)
        start_store(0, 0)

        wait_gather(1)
        start_gather(2, 0)
        transpose(1)
        start_store(1, 1)

        def steady(p, carry):
            k = 2 * p + 2
            wait_gather(0)
            start_gather(k + 1, 1)
            wait_store(0)
            transpose(0)
            start_store(k, 0)

            wait_gather(1)
            start_gather(k + 2, 0)
            wait_store(1)
            transpose(1)
            start_store(k + 1, 1)
            return carry

        lax.fori_loop(0, (nblk - 4) // 2, steady, 0)

        wait_gather(0)
        start_gather(nblk - 1, 1)
        wait_store(0)
        transpose(0)
        start_store(nblk - 2, 0)

        wait_gather(1)
        wait_store(1)
        transpose(1)
        start_store(nblk - 1, 1)

        wait_store(0)
        wait_store(1)

    return body(ids4, table)


def kernel(token_ids, table):
    b, s = token_ids.shape
    emb = table.shape[1]
    ids4 = token_ids.T.reshape(s // 8, 8, b // 128, 128).transpose(0, 2, 1, 3)
    y = _sc_embedding_lookup(ids4, table, b, s)
    y5 = y.reshape(s, emb // 8, b // 128, 8, 128)
    return y5.transpose(2, 4, 0, 1, 3).reshape(b, s, emb)


# speedup vs baseline: 1.1424x; 1.1424x over previous
import functools

import jax
import jax.numpy as jnp
from jax import lax
from jax.experimental import pallas as pl
from jax.experimental.pallas import tpu as pltpu
from jax.experimental.pallas import tpu_sc as plsc

_NC = 2
_NS = 16
_NW = _NC * _NS
_L = 16
_BB = 128
_EMB = 64


def _sc_embedding_lookup(ids4, table, b, s):
    nbt = b // _BB
    nst = s // 8
    assert nbt == _NW and nst * 8 == s and s % 2 == 0

    mesh = plsc.VectorSubcoreMesh(core_axis_name="c", subcore_axis_name="s")

    @functools.partial(
        pl.kernel,
        mesh=mesh,
        out_type=jax.ShapeDtypeStruct((s, _EMB // 8, nbt, 8 * _BB), jnp.float32),
        scratch_types=[
            pltpu.VMEM((nst, 8, _BB), jnp.int32),
            pltpu.VMEM((2, _BB, _EMB), jnp.float32),
            pltpu.VMEM((2, _BB * _EMB), jnp.float32),
            pltpu.SemaphoreType.DMA,
            pltpu.SemaphoreType.DMA,
            pltpu.SemaphoreType.DMA,
            pltpu.SemaphoreType.DMA,
        ],
        compiler_params=pltpu.CompilerParams(
            use_tc_tiling_on_sc=False, needs_layout_passes=False
        ),
    )
    def body(ids_hbm, table_hbm, y_hbm, idx_v, rows_v, yblk_v, g0, g1, s0, s1):
        wid = lax.axis_index("s") * _NC + lax.axis_index("c")
        gsems = (g0, g1)
        ssems = (s0, s1)

        def start_gather(k, buf):
            st = k // 8
            sr = k % 8
            pltpu.async_copy(
                table_hbm.at[idx_v.at[st, sr]], rows_v.at[buf], gsems[buf]
            )

        def wait_gather(buf):
            pltpu.make_async_copy(
                table_hbm.at[pl.ds(0, _BB)], rows_v.at[buf], gsems[buf]
            ).wait()

        def start_store(k, buf):
            for tc in range(_EMB // 8):
                pltpu.async_copy(
                    yblk_v.at[buf, pl.ds(tc * 8 * _BB, 8 * _BB)],
                    y_hbm.at[k, tc, wid],
                    ssems[buf],
                )

        def wait_store(buf):
            for tc in range(_EMB // 8):
                pltpu.make_async_copy(
                    yblk_v.at[buf, pl.ds(tc * 8 * _BB, 8 * _BB)],
                    y_hbm.at[0, tc, 0],
                    ssems[buf],
                ).wait()

        def transpose(buf):
            yb = yblk_v.at[buf]
            idx0 = lax.iota(jnp.int32, _L) * _BB

            def tbody(br, idx):
                for j in range(_EMB // _L):
                    vals = rows_v[buf, br, pl.ds(j * _L, _L)]
                    plsc.store_scatter(yb, [idx + (j * _L * _BB)], vals)
                return idx + 1

            lax.fori_loop(0, _BB, tbody, idx0)

        for st in range(nst):
            pltpu.sync_copy(ids_hbm.at[st, wid], idx_v.at[st])

        nblk = s
        start_gather(0, 0)

        wait_gather(0)
        start_gather(1, 1)
        transpose(0)
        start_store(0, 0)

        wait_gather(1)
        start_gather(2, 0)
        transpose(1)
        start_store(1, 1)

        def steady(p, carry):
            k = 2 * p + 2
            wait_gather(0)
            start_gather(k + 1, 1)
            wait_store(0)
            transpose(0)
            start_store(k, 0)

            wait_gather(1)
            start_gather(k + 2, 0)
            wait_store(1)
            transpose(1)
            start_store(k + 1, 1)
            return carry

        lax.fori_loop(0, (nblk - 4) // 2, steady, 0)

        wait_gather(0)
        start_gather(nblk - 1, 1)
        wait_store(0)
        transpose(0)
        start_store(nblk - 2, 0)

        wait_gather(1)
        wait_store(1)
        transpose(1)
        start_store(nblk - 1, 1)

        wait_store(0)
        wait_store(1)

    return body(ids4, table)


def kernel(token_ids, table):
    b, s = token_ids.shape
    emb = table.shape[1]
    ids4 = token_ids.T.reshape(s // 8, 8, b // 128, 128).transpose(0, 2, 1, 3)
    y = _sc_embedding_lookup(ids4, table, b, s)
    y5 = y.reshape(s, emb // 8, b // 128, 8, 128)
    return y5.transpose(2, 4, 0, 1, 3).reshape(b, s, emb)


# speedup vs baseline: 1.3905x; 1.2172x over previous
import functools

import jax
import jax.numpy as jnp
from jax import lax
from jax.experimental import pallas as pl
from jax.experimental.pallas import tpu as pltpu
from jax.experimental.pallas import tpu_sc as plsc

_NC = 2
_NS = 16
_NW = _NC * _NS
_L = 16
_BB = 128
_EMB = 64


def _sc_embedding_lookup(ids4, table, b, s):
    nbt = b // _BB
    nst = s // 8
    assert nbt == _NW and nst * 8 == s and s % 2 == 0

    mesh = plsc.VectorSubcoreMesh(core_axis_name="c", subcore_axis_name="s")

    @functools.partial(
        pl.kernel,
        mesh=mesh,
        out_type=jax.ShapeDtypeStruct((s, _EMB // 8, nbt, 8 * _BB), jnp.float32),
        scratch_types=[
            pltpu.VMEM((nst, 8, _BB), jnp.int32),
            pltpu.VMEM((2, _BB, _EMB), jnp.float32),
            pltpu.VMEM((2, _BB * _EMB), jnp.float32),
            pltpu.SemaphoreType.DMA,
            pltpu.SemaphoreType.DMA,
            pltpu.SemaphoreType.DMA,
            pltpu.SemaphoreType.DMA,
        ],
        compiler_params=pltpu.CompilerParams(
            use_tc_tiling_on_sc=False, needs_layout_passes=False
        ),
    )
    def body(ids_hbm, table_hbm, y_hbm, idx_v, rows_v, yblk_v, g0, g1, s0, s1):
        wid = lax.axis_index("s") * _NC + lax.axis_index("c")
        gsems = (g0, g1)
        ssems = (s0, s1)

        def start_gather(k, buf):
            st = k // 8
            sr = k % 8
            pltpu.async_copy(
                table_hbm.at[idx_v.at[st, sr]], rows_v.at[buf], gsems[buf]
            )

        def wait_gather(buf):
            pltpu.make_async_copy(
                table_hbm.at[pl.ds(0, _BB)], rows_v.at[buf], gsems[buf]
            ).wait()

        def start_store(k, buf):
            for tc in range(_EMB // 8):
                pltpu.async_copy(
                    yblk_v.at[buf, pl.ds(tc * 8 * _BB, 8 * _BB)],
                    y_hbm.at[k, tc, wid],
                    ssems[buf],
                )

        def wait_store(buf):
            for tc in range(_EMB // 8):
                pltpu.make_async_copy(
                    yblk_v.at[buf, pl.ds(tc * 8 * _BB, 8 * _BB)],
                    y_hbm.at[0, tc, 0],
                    ssems[buf],
                ).wait()

        def transpose(buf):
            yb = yblk_v.at[buf]
            idx0 = lax.iota(jnp.int32, _L) * _BB

            def tbody(br, idx):
                for j in range(_EMB // _L):
                    vals = rows_v[buf, br, pl.ds(j * _L, _L)]
                    plsc.store_scatter(yb, [idx + (j * _L * _BB)], vals)
                return idx + 1

            plsc.parallel_loop(0, _BB, step=1, unroll=8, carry=idx0)(tbody)

        for st in range(nst):
            pltpu.sync_copy(ids_hbm.at[st, wid], idx_v.at[st])

        nblk = s
        start_gather(0, 0)

        wait_gather(0)
        start_gather(1, 1)
        transpose(0)
        start_store(0, 0)

        wait_gather(1)
        start_gather(2, 0)
        transpose(1)
        start_store(1, 1)

        def steady(p, carry):
            k = 2 * p + 2
            wait_gather(0)
            start_gather(k + 1, 1)
            wait_store(0)
            transpose(0)
            start_store(k, 0)

            wait_gather(1)
            start_gather(k + 2, 0)
            wait_store(1)
            transpose(1)
            start_store(k + 1, 1)
            return carry

        lax.fori_loop(0, (nblk - 4) // 2, steady, 0)

        wait_gather(0)
        start_gather(nblk - 1, 1)
        wait_store(0)
        transpose(0)
        start_store(nblk - 2, 0)

        wait_gather(1)
        wait_store(1)
        transpose(1)
        start_store(nblk - 1, 1)

        wait_store(0)
        wait_store(1)

    return body(ids4, table)


def kernel(token_ids, table):
    b, s = token_ids.shape
    emb = table.shape[1]
    ids4 = token_ids.T.reshape(s // 8, 8, b // 128, 128).transpose(0, 2, 1, 3)
    y = _sc_embedding_lookup(ids4, table, b, s)
    y5 = y.reshape(s, emb // 8, b // 128, 8, 128)
    return y5.transpose(2, 4, 0, 1, 3).reshape(b, s, emb)


# speedup vs baseline: 2.1565x; 1.5509x over previous
import functools

import jax
import jax.numpy as jnp
from jax import lax
from jax.experimental import pallas as pl
from jax.experimental.pallas import tpu as pltpu
from jax.experimental.pallas import tpu_sc as plsc

_NC = 2
_NS = 16
_NW = _NC * _NS
_L = 16
_BB = 128
_EMB = 64


def _sc_embedding_lookup(ids4, table, b, s):
    nbt = b // _BB
    nst = s // 8
    assert nbt == _NW and nst * 8 == s and s % 2 == 0

    mesh = plsc.VectorSubcoreMesh(core_axis_name="c", subcore_axis_name="s")

    @functools.partial(
        pl.kernel,
        mesh=mesh,
        out_type=jax.ShapeDtypeStruct((s, _EMB // 8, nbt, 8, _BB), jnp.float32),
        scratch_types=[
            pltpu.VMEM((nst, 8, _BB), jnp.int32),
            pltpu.VMEM((2, _BB, _EMB), jnp.float32),
            pltpu.VMEM((2, _EMB, _BB + 1), jnp.float32),
            pltpu.SemaphoreType.DMA,
            pltpu.SemaphoreType.DMA,
            pltpu.SemaphoreType.DMA,
            pltpu.SemaphoreType.DMA,
        ],
        compiler_params=pltpu.CompilerParams(
            use_tc_tiling_on_sc=False, needs_layout_passes=False
        ),
    )
    def body(ids_hbm, table_hbm, y_hbm, idx_v, rows_v, yblk_v, g0, g1, s0, s1):
        wid = lax.axis_index("s") * _NC + lax.axis_index("c")
        gsems = (g0, g1)
        ssems = (s0, s1)

        def start_gather(k, buf):
            st = k // 8
            sr = k % 8
            pltpu.async_copy(
                table_hbm.at[idx_v.at[st, sr]], rows_v.at[buf], gsems[buf]
            )

        def wait_gather(buf):
            pltpu.make_async_copy(
                table_hbm.at[pl.ds(0, _BB)], rows_v.at[buf], gsems[buf]
            ).wait()

        def start_store(k, buf):
            for tc in range(_EMB // 8):
                pltpu.async_copy(
                    yblk_v.at[buf, pl.ds(tc * 8, 8), pl.ds(0, _BB)],
                    y_hbm.at[k, tc, wid],
                    ssems[buf],
                )

        def wait_store(buf):
            for tc in range(_EMB // 8):
                pltpu.make_async_copy(
                    yblk_v.at[buf, pl.ds(tc * 8, 8), pl.ds(0, _BB)],
                    y_hbm.at[0, tc, 0],
                    ssems[buf],
                ).wait()

        def transpose(buf):
            iota = lax.iota(jnp.int32, _L)
            cvecs = [iota + (j * _L) for j in range(_EMB // _L)]
            yb = yblk_v.at[buf]

            def tbody(br, brv):
                for j in range(_EMB // _L):
                    vals = rows_v[buf, br, pl.ds(j * _L, _L)]
                    plsc.store_scatter(yb, [cvecs[j], brv], vals)
                return brv + 1

            plsc.parallel_loop(
                0, _BB, step=1, unroll=8, carry=jnp.zeros((_L,), jnp.int32)
            )(tbody)

        for st in range(nst):
            pltpu.sync_copy(ids_hbm.at[st, wid], idx_v.at[st])

        nblk = s
        start_gather(0, 0)

        wait_gather(0)
        start_gather(1, 1)
        transpose(0)
        start_store(0, 0)

        wait_gather(1)
        start_gather(2, 0)
        transpose(1)
        start_store(1, 1)

        def steady(p, carry):
            k = 2 * p + 2
            wait_gather(0)
            start_gather(k + 1, 1)
            wait_store(0)
            transpose(0)
            start_store(k, 0)

            wait_gather(1)
            start_gather(k + 2, 0)
            wait_store(1)
            transpose(1)
            start_store(k + 1, 1)
            return carry

        lax.fori_loop(0, (nblk - 4) // 2, steady, 0)

        wait_gather(0)
        start_gather(nblk - 1, 1)
        wait_store(0)
        transpose(0)
        start_store(nblk - 2, 0)

        wait_gather(1)
        wait_store(1)
        transpose(1)
        start_store(nblk - 1, 1)

        wait_store(0)
        wait_store(1)

    return body(ids4, table)


def kernel(token_ids, table):
    b, s = token_ids.shape
    emb = table.shape[1]
    ids4 = token_ids.T.reshape(s // 8, 8, b // 128, 128).transpose(0, 2, 1, 3)
    y5 = _sc_embedding_lookup(ids4, table, b, s)
    return y5.transpose(2, 4, 0, 1, 3).reshape(b, s, emb)
